# baseline (speedup 1.0000x reference)
import functools

import jax
import jax.numpy as jnp
from jax import lax
from jax.experimental import pallas as pl
from jax.experimental.pallas import tpu as pltpu, tpu_sc as plsc

_ROWS_PER_WORKER = 400
_N_WORK = 25


def kernel(x, edge_index):
    del edge_index
    n, d = x.shape
    info = plsc.get_sparse_core_info()
    nc = info.num_cores
    mesh = plsc.VectorSubcoreMesh(core_axis_name="c", subcore_axis_name="s")

    @functools.partial(
        pl.kernel, mesh=mesh,
        out_type=jax.ShapeDtypeStruct((n, 4 * d), jnp.float32),
        scratch_types=[pltpu.VMEM((_ROWS_PER_WORKER, d), jnp.float32)],
    )
    def k(x_hbm, out_hbm, vbuf):
        wid = lax.axis_index("s") * nc + lax.axis_index("c")

        @pl.when(wid < _N_WORK)
        def _():
            base = wid * _ROWS_PER_WORKER
            rows = pl.ds(base, _ROWS_PER_WORKER)
            pltpu.sync_copy(x_hbm.at[rows, :], vbuf)
            for j in range(4):
                pltpu.sync_copy(vbuf, out_hbm.at[rows, pl.ds(j * d, d)])

    return k(x)


# speedup vs baseline: 3.3232x; 3.3232x over previous
import jax
import jax.numpy as jnp
from jax.experimental import pallas as pl
from jax.experimental.pallas import tpu as pltpu


_CHUNK_ROWS = (5000, 5000)
_CHUNKS = len(_CHUNK_ROWS)


def _dma_tile4_kernel(x_hbm, o_hbm, vbuf, in_sems, out_sems):
    n, d = vbuf.shape
    in_cps = []
    base = 0
    for c, h in enumerate(_CHUNK_ROWS):
        rows = pl.ds(base, h)
        base += h
        cp = pltpu.make_async_copy(
            x_hbm.at[rows, :], vbuf.at[rows, :], in_sems.at[c])
        cp.start()
        in_cps.append(cp)
    out_cps = []
    base = 0
    for c, h in enumerate(_CHUNK_ROWS):
        in_cps[c].wait()
        rows = pl.ds(base, h)
        base += h
        for j in range(4):
            cp = pltpu.make_async_copy(
                vbuf.at[rows, :], o_hbm.at[rows, pl.ds(j * d, d)],
                out_sems.at[c, j])
            cp.start()
            out_cps.append(cp)
    for cp in out_cps:
        cp.wait()


def kernel(x, edge_index):
    del edge_index
    n, d = x.shape
    out = pl.pallas_call(
        _dma_tile4_kernel,
        in_specs=[pl.BlockSpec(memory_space=pl.ANY)],
        out_specs=pl.BlockSpec(memory_space=pl.ANY),
        out_shape=jax.ShapeDtypeStruct((n, 4 * d), x.dtype),
        scratch_shapes=[
            pltpu.VMEM((n, d), x.dtype),
            pltpu.SemaphoreType.DMA((_CHUNKS,)),
            pltpu.SemaphoreType.DMA((_CHUNKS, 4)),
        ],
    )(x)
    return out
